# baseline (speedup 1.0000x reference)
import functools

import jax
import jax.numpy as jnp
from jax import lax
from jax.experimental import pallas as pl
from jax.experimental.pallas import tpu as pltpu
from jax.experimental.pallas import tpu_sc as plsc

_NUM_FIELDS = 26
_VOCAB1 = 100001
_DIM = 32
_BATCH = 4096

_NC = 2
_NS = 16
_NW = _NC * _NS

_ROWS = _NUM_FIELDS * _BATCH
_RPW = _ROWS // _NW
_CHUNK = 128
_NCH = _RPW // _CHUNK
_CH_PER_FIELD = _BATCH // _CHUNK


def _gather_body(idx_hbm, table_hbm, out_hbm, idx_v, rows_v, sem):
    wid = lax.axis_index("s") * _NC + lax.axis_index("c")
    chunk0 = wid * _NCH

    pltpu.sync_copy(idx_hbm.at[wid], idx_v)

    def add_offsets(j, carry):
        field = (chunk0 + j) // _CH_PER_FIELD
        off = field * _VOCAB1
        for g in range(_CHUNK // 16):
            sl = pl.ds(g * 16, 16)
            idx_v[j, sl] = idx_v[j, sl] + off
        return carry

    lax.fori_loop(0, _NCH, add_offsets, 0)

    copies = []
    for j in range(_NCH):
        copies.append(
            pltpu.async_copy(
                table_hbm.at[idx_v.at[j]],
                rows_v.at[pl.ds(j * _CHUNK, _CHUNK)],
                sem,
            )
        )
    for c in copies:
        c.wait()

    pltpu.sync_copy(rows_v, out_hbm.at[wid])


_mesh = plsc.VectorSubcoreMesh(core_axis_name="c", subcore_axis_name="s")

_gather = functools.partial(
    pl.kernel,
    out_type=jax.ShapeDtypeStruct((_NW, _RPW, _DIM), jnp.float32),
    mesh=_mesh,
    scratch_types=[
        pltpu.VMEM((_NCH, _CHUNK), jnp.int32),
        pltpu.VMEM((_RPW, _DIM), jnp.float32),
        pltpu.SemaphoreType.DMA,
    ],
    compiler_params=pltpu.CompilerParams(use_tc_tiling_on_sc=False),
)(_gather_body)


@jax.jit
def kernel(sparse_inputs, tables):
    idx = sparse_inputs.T.reshape(_NW, _NCH, _CHUNK)
    table_flat = tables.reshape(_NUM_FIELDS * _VOCAB1, _DIM)
    out = _gather(idx, table_flat)
    out = out.reshape(_NUM_FIELDS, _BATCH, _DIM)
    return tuple(out[i] for i in range(_NUM_FIELDS))


# speedup vs baseline: 18.9504x; 18.9504x over previous
import functools

import jax
import jax.numpy as jnp
from jax import lax
from jax.experimental import pallas as pl
from jax.experimental.pallas import tpu as pltpu
from jax.experimental.pallas import tpu_sc as plsc

_NUM_FIELDS = 26
_VOCAB1 = 100001
_DIM = 32
_BATCH = 4096

_NC = 2
_NS = 16
_DB = _DIM // 8
_SLABS = _NUM_FIELDS * _DB
_SLABS_PER_SC = _SLABS // _NC
_ROUNDS = _SLABS_PER_SC // 2
_GVEC = _BATCH // 16
_VMAIN = (_VOCAB1 // 128) * 128
_VTAIL = 128
_VCH = 8192
_NFULL = _VMAIN // _VCH
_VREM = _VMAIN - _NFULL * _VCH
_VSUB = _VCH // 8


def _lookup_body(idx_hbm, table_hbm, tail_hbm, out_hbm,
                 idx_sp, slab_sp, oslab_sp, idx_v, row_v, out_v):
    c = lax.axis_index("c")
    s = lax.axis_index("s")
    grp = s // 8
    sg = s % 8

    @pl.when(s == 0)
    def _():
        pltpu.sync_copy(idx_hbm, idx_sp)
    plsc.subcore_barrier()

    def round_body(r, carry):
        slab = c * _SLABS_PER_SC + 2 * r + grp
        f = slab // _DB
        d0 = pl.multiple_of((slab % _DB) * 8, 8)

        row_flat = row_v
        pltpu.sync_copy(idx_sp.at[f], idx_v)
        for k in range(_NFULL):
            off = k * _VCH
            pltpu.sync_copy(
                table_hbm.at[f, pl.ds(d0, 8), pl.ds(off + sg * _VSUB, _VSUB)],
                slab_sp.at[grp, :, pl.ds(sg * _VSUB, _VSUB)],
            )
            plsc.subcore_barrier()
            pltpu.sync_copy(slab_sp.at[grp, sg], row_flat.at[pl.ds(off, _VCH)])
            plsc.subcore_barrier()
        off = _NFULL * _VCH
        @pl.when(sg == 0)
        def _():
            pltpu.sync_copy(
                table_hbm.at[f, pl.ds(d0, 8), pl.ds(off, _VREM)],
                slab_sp.at[grp, :, pl.ds(0, _VREM)],
            )
        @pl.when(sg == 1)
        def _():
            pltpu.sync_copy(
                tail_hbm.at[f, pl.ds(d0, 8)],
                slab_sp.at[grp, :, pl.ds(_VREM, _VTAIL)],
            )
        plsc.subcore_barrier()
        pltpu.sync_copy(slab_sp.at[grp, sg, pl.ds(0, _VREM + _VTAIL)],
                        row_flat.at[pl.ds(off, _VREM + _VTAIL)])

        def g(i, carry2):
            sl = pl.ds(i * 16, 16)
            ii = jnp.minimum(jnp.maximum(idx_v[sl], 0), _VOCAB1 - 1)
            out_v[sl] = plsc.load_gather(row_v, [ii])
            return carry2

        lax.fori_loop(0, _GVEC, g, 0)

        pltpu.sync_copy(out_v, oslab_sp.at[grp, sg])
        plsc.subcore_barrier()

        @pl.when(sg == 0)
        def _():
            pltpu.sync_copy(oslab_sp.at[grp], out_hbm.at[f, pl.ds(d0, 8)])
        plsc.subcore_barrier()
        return carry

    lax.fori_loop(0, _ROUNDS, round_body, 0)


_mesh = plsc.VectorSubcoreMesh(core_axis_name="c", subcore_axis_name="s")

_lookup = functools.partial(
    pl.kernel,
    out_type=jax.ShapeDtypeStruct((_NUM_FIELDS, _DIM, _BATCH), jnp.float32),
    mesh=_mesh,
    scratch_types=[
        pltpu.VMEM_SHARED((32, _BATCH), jnp.int32),
        pltpu.VMEM_SHARED((2, 8, _VCH), jnp.float32),
        pltpu.VMEM_SHARED((2, 8, _BATCH), jnp.float32),
        pltpu.VMEM((_BATCH,), jnp.int32),
        pltpu.VMEM((_VMAIN + _VTAIL,), jnp.float32),
        pltpu.VMEM((_BATCH,), jnp.float32),
    ],
    compiler_params=pltpu.CompilerParams(needs_layout_passes=False),
)(_lookup_body)


@jax.jit
def kernel(sparse_inputs, tables):
    idx_t = jnp.pad(sparse_inputs.T, ((0, 32 - _NUM_FIELDS), (0, 0)))
    tab_t = jnp.transpose(tables, (0, 2, 1))
    tail = jnp.pad(tab_t[:, :, _VMAIN:], ((0, 0), (0, 0), (0, _VTAIL - (_VOCAB1 - _VMAIN))))
    out = _lookup(idx_t, tab_t, tail)
    return tuple(out[i].T for i in range(_NUM_FIELDS))
